# initial kernel scaffold (unmeasured)
import jax
import jax.numpy as jnp
from jax import lax
from jax.experimental import pallas as pl
from jax.experimental.pallas import tpu as pltpu

N_DEV = 4
SQ = 2048
SKV = 2048
DH = 128
HG = 8
DM = 1024
QC = 512
N_QC = SQ // QC
SCALE = 0.08838834764831843
NEG = -1e9


def kernel(x, Wq, K_ext, V_ext, Wo):
    my = lax.axis_index("i")
    xb = x[0].astype(jnp.bfloat16)
    wq = Wq.astype(jnp.bfloat16)
    wo = Wo.astype(jnp.bfloat16)
    k_loc = lax.dynamic_index_in_dim(K_ext, my, 0, keepdims=False)
    v_loc = lax.dynamic_index_in_dim(V_ext, my, 0, keepdims=False)
    kt = jnp.transpose(k_loc.astype(jnp.bfloat16), (1, 0, 2))
    vt = jnp.transpose(v_loc.astype(jnp.bfloat16), (1, 0, 2))

    def body(x_ref, wq_ref, wo_ref, kt_ref, vt_ref, out_ref,
             comm_wq, comm_wo, q_scr, ctx_scr, kg, vg, bias_scr,
             send_sems, recv_sems, credit_sem, kv_sems):
        my_i = lax.axis_index("i")
        left = lax.rem(my_i + N_DEV - 1, N_DEV)
        right = lax.rem(my_i + 1, N_DEV)

        def bias_body(qc, carry):
            r0 = qc * QC
            qblk = (r0 + lax.broadcasted_iota(jnp.int32, (QC, SKV), 0)) // 64
            kblk = lax.broadcasted_iota(jnp.int32, (QC, SKV), 1) // 64
            keep = (qblk == kblk) | (kblk == 0) | (lax.rem(qblk + kblk, 3) == 0)
            bias_scr[pl.ds(r0, QC), :] = jnp.where(keep, 0.0, NEG).astype(
                jnp.bfloat16)
            return carry
        lax.fori_loop(0, N_QC, bias_body, 0)

        barrier = pltpu.get_barrier_semaphore()
        for nbr in (left, right):
            pl.semaphore_signal(barrier, inc=1, device_id=(nbr,),
                                device_id_type=pl.DeviceIdType.MESH)
        pl.semaphore_wait(barrier, 2)

        rdmas = []
        for j in range(N_DEV):
            o = lax.rem(my_i - j + N_DEV, N_DEV)

            kcp = pltpu.make_async_copy(kt_ref.at[pl.ds(o * HG, HG)], kg,
                                        kv_sems.at[0])
            vcp = pltpu.make_async_copy(vt_ref.at[pl.ds(o * HG, HG)], vg,
                                        kv_sems.at[1])
            kcp.start()
            vcp.start()

            if j < N_DEV - 1:
                if j == 2:
                    pl.semaphore_wait(credit_sem, 1)
                src_wq = wq_ref if j == 0 else comm_wq.at[(j - 1) % 2]
                src_wo = wo_ref if j == 0 else comm_wo.at[(j - 1) % 2]
                r_wq = pltpu.make_async_remote_copy(
                    src_ref=src_wq, dst_ref=comm_wq.at[j % 2],
                    send_sem=send_sems.at[j, 0], recv_sem=recv_sems.at[j, 0],
                    device_id=(right,), device_id_type=pl.DeviceIdType.MESH)
                r_wo = pltpu.make_async_remote_copy(
                    src_ref=src_wo, dst_ref=comm_wo.at[j % 2],
                    send_sem=send_sems.at[j, 1], recv_sem=recv_sems.at[j, 1],
                    device_id=(right,), device_id_type=pl.DeviceIdType.MESH)
                r_wq.start()
                r_wo.start()
                rdmas.append((r_wq, r_wo))

            wq_j = wq_ref[...] if j == 0 else comm_wq[(j - 1) % 2]
            q_scr[...] = (jnp.dot(x_ref[...], wq_j,
                                  preferred_element_type=jnp.float32)
                          * SCALE).astype(jnp.bfloat16)
            kcp.wait()
            vcp.wait()

            def qc_body(qc, carry):
                def h_body(h, c2):
                    q_blk = q_scr[pl.ds(qc * QC, QC), pl.ds(h * DH, DH)]
                    s = lax.dot_general(
                        q_blk, kg[h], (((1,), (1,)), ((), ())),
                        preferred_element_type=jnp.float32)
                    s = s + bias_scr[pl.ds(qc * QC, QC), :].astype(jnp.float32)
                    mx = jnp.max(s, axis=1, keepdims=True)
                    w = jnp.exp(s - mx)
                    sm = jnp.sum(w, axis=1, keepdims=True)
                    wb = (w / sm).astype(jnp.bfloat16)
                    ctx_scr[:, pl.ds(h * DH, DH)] = lax.dot_general(
                        wb, vg[h], (((1,), (0,)), ((), ())),
                        preferred_element_type=jnp.float32).astype(jnp.bfloat16)
                    return c2
                lax.fori_loop(0, HG, h_body, 0)
                wo_j = wo_ref[...] if j == 0 else comm_wo[(j - 1) % 2]
                oval = jnp.dot(ctx_scr[...], wo_j,
                               preferred_element_type=jnp.float32)
                if j == 0:
                    out_ref[0, pl.ds(qc * QC, QC), :] = oval
                else:
                    out_ref[0, pl.ds(qc * QC, QC), :] = (
                        out_ref[0, pl.ds(qc * QC, QC), :] + oval)
                return carry
            lax.fori_loop(0, N_QC, qc_body, 0)

            if j < N_DEV - 1:
                r_wq, r_wo = rdmas[j]
                r_wq.wait_send()
                r_wo.wait_send()
                if j == 1:
                    pl.semaphore_signal(credit_sem, inc=1, device_id=(left,),
                                        device_id_type=pl.DeviceIdType.MESH)
                r_wq.wait_recv()
                r_wo.wait_recv()

    return pl.pallas_call(
        body,
        out_shape=jax.ShapeDtypeStruct((1, SQ, DM), jnp.float32),
        in_specs=[
            pl.BlockSpec(memory_space=pltpu.VMEM),
            pl.BlockSpec(memory_space=pltpu.VMEM),
            pl.BlockSpec(memory_space=pltpu.VMEM),
            pl.BlockSpec(memory_space=pltpu.ANY),
            pl.BlockSpec(memory_space=pltpu.ANY),
        ],
        out_specs=pl.BlockSpec(memory_space=pltpu.VMEM),
        scratch_shapes=[
            pltpu.VMEM((2, DM, DM), jnp.bfloat16),
            pltpu.VMEM((2, DM, DM), jnp.bfloat16),
            pltpu.VMEM((SQ, DM), jnp.bfloat16),
            pltpu.VMEM((QC, DM), jnp.bfloat16),
            pltpu.VMEM((HG, SKV, DH), jnp.bfloat16),
            pltpu.VMEM((HG, SKV, DH), jnp.bfloat16),
            pltpu.VMEM((SQ, SKV), jnp.bfloat16),
            pltpu.SemaphoreType.DMA((N_DEV - 1, 2)),
            pltpu.SemaphoreType.DMA((N_DEV - 1, 2)),
            pltpu.SemaphoreType.REGULAR,
            pltpu.SemaphoreType.DMA((2,)),
        ],
        compiler_params=pltpu.CompilerParams(collective_id=0),
    )(xb, wq, wo, kt, vt)


# baseline (device time: 475109 ns/iter reference)
import jax
import jax.numpy as jnp
from jax import lax
from jax.experimental import pallas as pl
from jax.experimental.pallas import tpu as pltpu

N_DEV = 4
SQ = 2048
SKV = 2048
DH = 128
HG = 8
DM = 1024
QC = 256
N_QC = SQ // QC
SCALE = 0.08838834764831843
NEG = -1e9


def kernel(x, Wq, K_ext, V_ext, Wo):
    my = lax.axis_index("i")
    xb = x[0].astype(jnp.bfloat16)
    wq = Wq.astype(jnp.bfloat16)
    wo = Wo.astype(jnp.bfloat16)
    k_loc = lax.dynamic_index_in_dim(K_ext, my, 0, keepdims=False)
    v_loc = lax.dynamic_index_in_dim(V_ext, my, 0, keepdims=False)
    kt = jnp.transpose(k_loc.astype(jnp.bfloat16), (1, 0, 2))
    vt = jnp.transpose(v_loc.astype(jnp.bfloat16), (1, 0, 2))

    def body(x_ref, wq_ref, wo_ref, kt_ref, vt_ref, out_ref,
             comm_wq, comm_wo, q_scr, ctx_scr, kg, vg, bias_scr,
             send_sems, recv_sems, credit_sem, kv_sems):
        my_i = lax.axis_index("i")
        left = lax.rem(my_i + N_DEV - 1, N_DEV)
        right = lax.rem(my_i + 1, N_DEV)

        def bias_body(qc, carry):
            r0 = qc * QC
            qblk = (r0 + lax.broadcasted_iota(jnp.int32, (QC, SKV), 0)) // 64
            kblk = lax.broadcasted_iota(jnp.int32, (QC, SKV), 1) // 64
            keep = (qblk == kblk) | (kblk == 0) | (lax.rem(qblk + kblk, 3) == 0)
            bias_scr[pl.ds(r0, QC), :] = jnp.where(keep, 0.0, NEG).astype(
                jnp.bfloat16)
            return carry
        lax.fori_loop(0, N_QC, bias_body, 0)

        barrier = pltpu.get_barrier_semaphore()
        for nbr in (left, right):
            pl.semaphore_signal(barrier, inc=1, device_id=(nbr,),
                                device_id_type=pl.DeviceIdType.MESH)
        pl.semaphore_wait(barrier, 2)

        rdmas = []
        for j in range(N_DEV):
            o = lax.rem(my_i - j + N_DEV, N_DEV)

            kcp = pltpu.make_async_copy(kt_ref.at[pl.ds(o * HG, HG)], kg,
                                        kv_sems.at[0])
            vcp = pltpu.make_async_copy(vt_ref.at[pl.ds(o * HG, HG)], vg,
                                        kv_sems.at[1])
            kcp.start()
            vcp.start()

            if j < N_DEV - 1:
                if j == 2:
                    pl.semaphore_wait(credit_sem, 1)
                src_wq = wq_ref if j == 0 else comm_wq.at[(j - 1) % 2]
                src_wo = wo_ref if j == 0 else comm_wo.at[(j - 1) % 2]
                r_wq = pltpu.make_async_remote_copy(
                    src_ref=src_wq, dst_ref=comm_wq.at[j % 2],
                    send_sem=send_sems.at[j, 0], recv_sem=recv_sems.at[j, 0],
                    device_id=(right,), device_id_type=pl.DeviceIdType.MESH)
                r_wo = pltpu.make_async_remote_copy(
                    src_ref=src_wo, dst_ref=comm_wo.at[j % 2],
                    send_sem=send_sems.at[j, 1], recv_sem=recv_sems.at[j, 1],
                    device_id=(right,), device_id_type=pl.DeviceIdType.MESH)
                r_wq.start()
                r_wo.start()
                rdmas.append((r_wq, r_wo))

            def qproj_body(qc, carry):
                wq_j = wq_ref[...] if j == 0 else comm_wq[(j - 1) % 2]
                q_scr[pl.ds(qc * QC, QC), :] = (
                    jnp.dot(x_ref[pl.ds(qc * QC, QC), :], wq_j,
                            preferred_element_type=jnp.float32)
                    * SCALE).astype(jnp.bfloat16)
                return carry
            lax.fori_loop(0, N_QC, qproj_body, 0)
            kcp.wait()
            vcp.wait()

            def qc_body(qc, carry):
                def h_body(h, c2):
                    q_blk = q_scr[pl.ds(qc * QC, QC), pl.ds(h * DH, DH)]
                    s = lax.dot_general(
                        q_blk, kg[h], (((1,), (1,)), ((), ())),
                        preferred_element_type=jnp.float32)
                    s = s + bias_scr[pl.ds(qc * QC, QC), :].astype(jnp.float32)
                    mx = jnp.max(s, axis=1, keepdims=True)
                    w = jnp.exp(s - mx)
                    sm = jnp.sum(w, axis=1, keepdims=True)
                    wb = (w / sm).astype(jnp.bfloat16)
                    ctx_scr[:, pl.ds(h * DH, DH)] = lax.dot_general(
                        wb, vg[h], (((1,), (0,)), ((), ())),
                        preferred_element_type=jnp.float32).astype(jnp.bfloat16)
                    return c2
                lax.fori_loop(0, HG, h_body, 0)
                wo_j = wo_ref[...] if j == 0 else comm_wo[(j - 1) % 2]
                oval = jnp.dot(ctx_scr[...], wo_j,
                               preferred_element_type=jnp.float32)
                if j == 0:
                    out_ref[0, pl.ds(qc * QC, QC), :] = oval
                else:
                    out_ref[0, pl.ds(qc * QC, QC), :] = (
                        out_ref[0, pl.ds(qc * QC, QC), :] + oval)
                return carry
            lax.fori_loop(0, N_QC, qc_body, 0)

            if j < N_DEV - 1:
                r_wq, r_wo = rdmas[j]
                r_wq.wait_send()
                r_wo.wait_send()
                if j == 1:
                    pl.semaphore_signal(credit_sem, inc=1, device_id=(left,),
                                        device_id_type=pl.DeviceIdType.MESH)
                r_wq.wait_recv()
                r_wo.wait_recv()

    return pl.pallas_call(
        body,
        out_shape=jax.ShapeDtypeStruct((1, SQ, DM), jnp.float32),
        in_specs=[
            pl.BlockSpec(memory_space=pltpu.VMEM),
            pl.BlockSpec(memory_space=pltpu.VMEM),
            pl.BlockSpec(memory_space=pltpu.VMEM),
            pl.BlockSpec(memory_space=pl.ANY),
            pl.BlockSpec(memory_space=pl.ANY),
        ],
        out_specs=pl.BlockSpec(memory_space=pltpu.VMEM),
        scratch_shapes=[
            pltpu.VMEM((2, DM, DM), jnp.bfloat16),
            pltpu.VMEM((2, DM, DM), jnp.bfloat16),
            pltpu.VMEM((SQ, DM), jnp.bfloat16),
            pltpu.VMEM((QC, DM), jnp.bfloat16),
            pltpu.VMEM((HG, SKV, DH), jnp.bfloat16),
            pltpu.VMEM((HG, SKV, DH), jnp.bfloat16),
            pltpu.VMEM((SQ, SKV), jnp.bfloat16),
            pltpu.SemaphoreType.DMA((N_DEV - 1, 2)),
            pltpu.SemaphoreType.DMA((N_DEV - 1, 2)),
            pltpu.SemaphoreType.REGULAR,
            pltpu.SemaphoreType.DMA((2,)),
        ],
        compiler_params=pltpu.CompilerParams(collective_id=0),
    )(xb, wq, wo, kt, vt)


# device time: 302880 ns/iter; 1.5686x vs baseline; 1.5686x over previous
import jax
import jax.numpy as jnp
from jax import lax
from jax.experimental import pallas as pl
from jax.experimental.pallas import tpu as pltpu

N_DEV = 4
SQ = 2048
SKV = 2048
DH = 128
HG = 8
DM = 1024
QC = 512
N_QC = SQ // QC
SCALE = 0.08838834764831843
NEG = -1e9


def kernel(x, Wq, K_ext, V_ext, Wo):
    my = lax.axis_index("i")
    xb = x[0].astype(jnp.bfloat16)
    wq = Wq.astype(jnp.bfloat16)
    wo = Wo.astype(jnp.bfloat16)
    k_loc = lax.dynamic_index_in_dim(K_ext, my, 0, keepdims=False)
    v_loc = lax.dynamic_index_in_dim(V_ext, my, 0, keepdims=False)
    kt = jnp.transpose(k_loc.astype(jnp.bfloat16), (1, 0, 2))
    vt = jnp.transpose(v_loc.astype(jnp.bfloat16), (1, 0, 2))

    def body(x_ref, wq_ref, wo_ref, kt_ref, vt_ref, out_ref,
             comm_wq, comm_wo, q_scr, ctx_scr, kg, vg, bias_scr,
             send_sems, recv_sems, credit_sem, kv_sems):
        my_i = lax.axis_index("i")
        left = lax.rem(my_i + N_DEV - 1, N_DEV)
        right = lax.rem(my_i + 1, N_DEV)

        def bias_body(qc, carry):
            r0 = qc * QC
            qblk = (r0 + lax.broadcasted_iota(jnp.int32, (QC, SKV), 0)) // 64
            kblk = lax.broadcasted_iota(jnp.int32, (QC, SKV), 1) // 64
            keep = (qblk == kblk) | (kblk == 0) | (lax.rem(qblk + kblk, 3) == 0)
            bias_scr[pl.ds(r0, QC), :] = jnp.where(keep, 0.0, NEG).astype(
                jnp.bfloat16)
            return carry
        lax.fori_loop(0, N_QC, bias_body, 0)

        barrier = pltpu.get_barrier_semaphore()
        for nbr in (left, right):
            pl.semaphore_signal(barrier, inc=1, device_id=(nbr,),
                                device_id_type=pl.DeviceIdType.MESH)
        pl.semaphore_wait(barrier, 2)

        rdmas = []
        for j in range(N_DEV):
            o = lax.rem(my_i - j + N_DEV, N_DEV)

            kcp = pltpu.make_async_copy(kt_ref.at[pl.ds(o * HG, HG)], kg,
                                        kv_sems.at[0])
            vcp = pltpu.make_async_copy(vt_ref.at[pl.ds(o * HG, HG)], vg,
                                        kv_sems.at[1])
            kcp.start()
            vcp.start()

            if j < N_DEV - 1:
                if j == 2:
                    pl.semaphore_wait(credit_sem, 1)
                src_wq = wq_ref if j == 0 else comm_wq.at[(j - 1) % 2]
                src_wo = wo_ref if j == 0 else comm_wo.at[(j - 1) % 2]
                r_wq = pltpu.make_async_remote_copy(
                    src_ref=src_wq, dst_ref=comm_wq.at[j % 2],
                    send_sem=send_sems.at[j, 0], recv_sem=recv_sems.at[j, 0],
                    device_id=(right,), device_id_type=pl.DeviceIdType.MESH)
                r_wo = pltpu.make_async_remote_copy(
                    src_ref=src_wo, dst_ref=comm_wo.at[j % 2],
                    send_sem=send_sems.at[j, 1], recv_sem=recv_sems.at[j, 1],
                    device_id=(right,), device_id_type=pl.DeviceIdType.MESH)
                r_wq.start()
                r_wo.start()
                rdmas.append((r_wq, r_wo))

            def qproj_body(qc, carry):
                wq_j = wq_ref[...] if j == 0 else comm_wq[(j - 1) % 2]
                q_scr[pl.ds(qc * QC, QC), :] = (
                    jnp.dot(x_ref[pl.ds(qc * QC, QC), :], wq_j,
                            preferred_element_type=jnp.float32)
                    * SCALE).astype(jnp.bfloat16)
                return carry
            lax.fori_loop(0, N_QC, qproj_body, 0)
            kcp.wait()
            vcp.wait()

            def qc_body(qc, carry):
                def h_body(h, c2):
                    q_blk = q_scr[pl.ds(qc * QC, QC), pl.ds(h * DH, DH)]
                    s = lax.dot_general(
                        q_blk, kg[h], (((1,), (1,)), ((), ())),
                        preferred_element_type=jnp.float32)
                    s = s + bias_scr[pl.ds(qc * QC, QC), :].astype(jnp.float32)
                    w = jnp.exp(s)
                    sm = jnp.sum(w, axis=1, keepdims=True)
                    ctx = lax.dot_general(
                        w.astype(jnp.bfloat16), vg[h], (((1,), (0,)), ((), ())),
                        preferred_element_type=jnp.float32)
                    ctx_scr[:, pl.ds(h * DH, DH)] = (
                        ctx * (1.0 / sm)).astype(jnp.bfloat16)
                    return c2
                lax.fori_loop(0, HG, h_body, 0)
                wo_j = wo_ref[...] if j == 0 else comm_wo[(j - 1) % 2]
                oval = jnp.dot(ctx_scr[...], wo_j,
                               preferred_element_type=jnp.float32)
                if j == 0:
                    out_ref[0, pl.ds(qc * QC, QC), :] = oval
                else:
                    out_ref[0, pl.ds(qc * QC, QC), :] = (
                        out_ref[0, pl.ds(qc * QC, QC), :] + oval)
                return carry
            lax.fori_loop(0, N_QC, qc_body, 0)

            if j < N_DEV - 1:
                r_wq, r_wo = rdmas[j]
                r_wq.wait_send()
                r_wo.wait_send()
                if j == 1:
                    pl.semaphore_signal(credit_sem, inc=1, device_id=(left,),
                                        device_id_type=pl.DeviceIdType.MESH)
                r_wq.wait_recv()
                r_wo.wait_recv()

    return pl.pallas_call(
        body,
        out_shape=jax.ShapeDtypeStruct((1, SQ, DM), jnp.float32),
        in_specs=[
            pl.BlockSpec(memory_space=pltpu.VMEM),
            pl.BlockSpec(memory_space=pltpu.VMEM),
            pl.BlockSpec(memory_space=pltpu.VMEM),
            pl.BlockSpec(memory_space=pl.ANY),
            pl.BlockSpec(memory_space=pl.ANY),
        ],
        out_specs=pl.BlockSpec(memory_space=pltpu.VMEM),
        scratch_shapes=[
            pltpu.VMEM((2, DM, DM), jnp.bfloat16),
            pltpu.VMEM((2, DM, DM), jnp.bfloat16),
            pltpu.VMEM((SQ, DM), jnp.bfloat16),
            pltpu.VMEM((QC, DM), jnp.bfloat16),
            pltpu.VMEM((HG, SKV, DH), jnp.bfloat16),
            pltpu.VMEM((HG, SKV, DH), jnp.bfloat16),
            pltpu.VMEM((SQ, SKV), jnp.bfloat16),
            pltpu.SemaphoreType.DMA((N_DEV - 1, 2)),
            pltpu.SemaphoreType.DMA((N_DEV - 1, 2)),
            pltpu.SemaphoreType.REGULAR,
            pltpu.SemaphoreType.DMA((2,)),
        ],
        compiler_params=pltpu.CompilerParams(collective_id=0),
    )(xb, wq, wo, kt, vt)


# device time: 293560 ns/iter; 1.6184x vs baseline; 1.0317x over previous
import jax
import jax.numpy as jnp
from jax import lax
from jax.experimental import pallas as pl
from jax.experimental.pallas import tpu as pltpu

N_DEV = 4
SQ = 2048
SKV = 2048
DH = 128
HG = 8
DM = 1024
QC = 256
N_QC = SQ // QC
SCALE = 0.08838834764831843
NEG = -1e9


def kernel(x, Wq, K_ext, V_ext, Wo):
    my = lax.axis_index("i")
    xb = x[0].astype(jnp.bfloat16)
    wq = Wq.astype(jnp.bfloat16)
    wo = Wo.astype(jnp.bfloat16)
    k_loc = lax.dynamic_index_in_dim(K_ext, my, 0, keepdims=False)
    v_loc = lax.dynamic_index_in_dim(V_ext, my, 0, keepdims=False)
    kt = jnp.transpose(k_loc.astype(jnp.bfloat16), (1, 0, 2))
    vt = jnp.transpose(v_loc.astype(jnp.bfloat16), (1, 0, 2))

    def body(x_ref, wq_ref, wo_ref, kt_ref, vt_ref, out_ref,
             comm_wq, comm_wo, q_scr, ctx_scr, kg, vg, bias_scr,
             send_sems, recv_sems, credit_sem, kv_sems):
        my_i = lax.axis_index("i")
        left = lax.rem(my_i + N_DEV - 1, N_DEV)
        right = lax.rem(my_i + 1, N_DEV)

        def bias_body(qc, carry):
            r0 = qc * QC
            qblk = (r0 + lax.broadcasted_iota(jnp.int32, (QC, SKV), 0)) // 64
            kblk = lax.broadcasted_iota(jnp.int32, (QC, SKV), 1) // 64
            keep = (qblk == kblk) | (kblk == 0) | (lax.rem(qblk + kblk, 3) == 0)
            bias_scr[pl.ds(r0, QC), :] = jnp.where(keep, 0.0, NEG).astype(
                jnp.bfloat16)
            return carry
        lax.fori_loop(0, N_QC, bias_body, 0)

        barrier = pltpu.get_barrier_semaphore()
        for nbr in (left, right):
            pl.semaphore_signal(barrier, inc=1, device_id=(nbr,),
                                device_id_type=pl.DeviceIdType.MESH)
        pl.semaphore_wait(barrier, 2)

        rdmas = []
        for j in range(N_DEV):
            o = lax.rem(my_i - j + N_DEV, N_DEV)

            kcp = pltpu.make_async_copy(kt_ref.at[pl.ds(o * HG, HG)], kg,
                                        kv_sems.at[0])
            vcp = pltpu.make_async_copy(vt_ref.at[pl.ds(o * HG, HG)], vg,
                                        kv_sems.at[1])
            kcp.start()
            vcp.start()

            if j < N_DEV - 1:
                if j == 2:
                    pl.semaphore_wait(credit_sem, 1)
                src_wq = wq_ref if j == 0 else comm_wq.at[(j - 1) % 2]
                src_wo = wo_ref if j == 0 else comm_wo.at[(j - 1) % 2]
                r_wq = pltpu.make_async_remote_copy(
                    src_ref=src_wq, dst_ref=comm_wq.at[j % 2],
                    send_sem=send_sems.at[j, 0], recv_sem=recv_sems.at[j, 0],
                    device_id=(right,), device_id_type=pl.DeviceIdType.MESH)
                r_wo = pltpu.make_async_remote_copy(
                    src_ref=src_wo, dst_ref=comm_wo.at[j % 2],
                    send_sem=send_sems.at[j, 1], recv_sem=recv_sems.at[j, 1],
                    device_id=(right,), device_id_type=pl.DeviceIdType.MESH)
                r_wq.start()
                r_wo.start()
                rdmas.append((r_wq, r_wo))

            def qproj_body(qc, carry):
                wq_j = wq_ref[...] if j == 0 else comm_wq[(j - 1) % 2]
                q_scr[pl.ds(qc * QC, QC), :] = (
                    jnp.dot(x_ref[pl.ds(qc * QC, QC), :], wq_j,
                            preferred_element_type=jnp.float32)
                    * SCALE).astype(jnp.bfloat16)
                return carry
            lax.fori_loop(0, N_QC, qproj_body, 0)
            kcp.wait()
            vcp.wait()

            def qc_body(qc, carry):
                bias_c = bias_scr[pl.ds(qc * QC, QC), :].astype(jnp.float32)
                for h in range(HG):
                    q_blk = q_scr[pl.ds(qc * QC, QC), pl.ds(h * DH, DH)]
                    s = lax.dot_general(
                        q_blk, kg[h], (((1,), (1,)), ((), ())),
                        preferred_element_type=jnp.float32)
                    w = jnp.exp(s + bias_c)
                    sm = jnp.sum(w, axis=1, keepdims=True)
                    ctx = lax.dot_general(
                        w.astype(jnp.bfloat16), vg[h], (((1,), (0,)), ((), ())),
                        preferred_element_type=jnp.float32)
                    ctx_scr[:, pl.ds(h * DH, DH)] = (
                        ctx * (1.0 / sm)).astype(jnp.bfloat16)
                wo_j = wo_ref[...] if j == 0 else comm_wo[(j - 1) % 2]
                oval = jnp.dot(ctx_scr[...], wo_j,
                               preferred_element_type=jnp.float32)
                if j == 0:
                    out_ref[0, pl.ds(qc * QC, QC), :] = oval
                else:
                    out_ref[0, pl.ds(qc * QC, QC), :] = (
                        out_ref[0, pl.ds(qc * QC, QC), :] + oval)
                return carry
            lax.fori_loop(0, N_QC, qc_body, 0)

            if j < N_DEV - 1:
                r_wq, r_wo = rdmas[j]
                r_wq.wait_send()
                r_wo.wait_send()
                if j == 1:
                    pl.semaphore_signal(credit_sem, inc=1, device_id=(left,),
                                        device_id_type=pl.DeviceIdType.MESH)
                r_wq.wait_recv()
                r_wo.wait_recv()

    return pl.pallas_call(
        body,
        out_shape=jax.ShapeDtypeStruct((1, SQ, DM), jnp.float32),
        in_specs=[
            pl.BlockSpec(memory_space=pltpu.VMEM),
            pl.BlockSpec(memory_space=pltpu.VMEM),
            pl.BlockSpec(memory_space=pltpu.VMEM),
            pl.BlockSpec(memory_space=pl.ANY),
            pl.BlockSpec(memory_space=pl.ANY),
        ],
        out_specs=pl.BlockSpec(memory_space=pltpu.VMEM),
        scratch_shapes=[
            pltpu.VMEM((2, DM, DM), jnp.bfloat16),
            pltpu.VMEM((2, DM, DM), jnp.bfloat16),
            pltpu.VMEM((SQ, DM), jnp.bfloat16),
            pltpu.VMEM((QC, DM), jnp.bfloat16),
            pltpu.VMEM((HG, SKV, DH), jnp.bfloat16),
            pltpu.VMEM((HG, SKV, DH), jnp.bfloat16),
            pltpu.VMEM((SQ, SKV), jnp.bfloat16),
            pltpu.SemaphoreType.DMA((N_DEV - 1, 2)),
            pltpu.SemaphoreType.DMA((N_DEV - 1, 2)),
            pltpu.SemaphoreType.REGULAR,
            pltpu.SemaphoreType.DMA((2,)),
        ],
        compiler_params=pltpu.CompilerParams(
            collective_id=0),
    )(xb, wq, wo, kt, vt)
